# baseline (device time: 92213 ns/iter reference)
import jax
import jax.numpy as jnp
from jax import lax
from jax.experimental import pallas as pl
from jax.experimental.pallas import tpu as pltpu

N_DEV = 4
SQ = 1024
SKV_LOC = 1024
HQ = 8
DH = 128
DM = 1024
SCALE = 0.08838834764831843
NEG = -1e9

HALF = SQ // 2
R0 = 32
R2 = 896
BF = jnp.bfloat16
F32 = jnp.float32


def _attn_body(q_ref, k_ref, v_ref, ctx_ref, stats_ref):
    my_pos = lax.axis_index("i")

    qi = lax.broadcasted_iota(jnp.int32, (SQ, SKV_LOC), 0)
    kj = lax.broadcasted_iota(jnp.int32, (SQ, SKV_LOC), 1) + my_pos * SKV_LOC
    mask = (jnp.abs(qi - kj) <= 128) | (kj < 32) | (qi < 32)

    m_parts = []
    l_parts = []
    for h in range(HQ):
        qh = q_ref[:, h * DH:(h + 1) * DH].astype(BF)
        kh = k_ref[0, :, h, :].astype(BF)
        vh = v_ref[0, :, h, :].astype(BF)
        s = lax.dot_general(qh, kh, (((1,), (1,)), ((), ())),
                            preferred_element_type=F32) * SCALE
        s = jnp.where(mask, s, NEG)
        m = jnp.max(s, axis=1, keepdims=True)
        w = jnp.exp(s - m)
        l = jnp.sum(w, axis=1, keepdims=True)
        ctx_ref[:, h * DH:(h + 1) * DH] = jnp.dot(
            w.astype(BF), vh, preferred_element_type=F32).astype(BF)
        m_parts.append(m)
        l_parts.append(l)

    stats_ref[...] = jnp.concatenate(m_parts + l_parts, axis=1)


def _merge_body(ctx_ref, stats_ref, out_ref,
                big, sm32, sm128, st_big, st32, st128,
                big_recv, sm32_recv, sm128_recv, st_big_recv, st32_recv,
                send_sems):
    my_pos = lax.axis_index("i")

    def rdma(src, dst, ssem, rsem, dev):
        return pltpu.make_async_remote_copy(
            src_ref=src, dst_ref=dst, send_sem=ssem, recv_sem=rsem,
            device_id=(dev,), device_id_type=pl.DeviceIdType.MESH)

    a_lo = ctx_ref.at[pl.ds(0, HALF)]
    a_hi = ctx_ref.at[pl.ds(HALF, HALF)]
    b_lo = big.at[pl.ds(0, HALF)]
    b_hi = big.at[pl.ds(HALF, HALF)]

    @pl.when(my_pos == 0)
    def _():
        sends = [
            rdma(a_lo, b_lo, send_sems.at[0], big_recv.at[0], 1),
            rdma(a_hi, b_hi, send_sems.at[1], big_recv.at[1], 3),
            rdma(a_hi, b_hi, send_sems.at[2], big_recv.at[1], 1),
            rdma(a_lo, b_lo, send_sems.at[3], big_recv.at[0], 3),
            rdma(stats_ref, st_big, send_sems.at[4], st_big_recv.at[0], 1),
            rdma(stats_ref, st_big, send_sems.at[5], st_big_recv.at[0], 2),
            rdma(stats_ref, st_big, send_sems.at[6], st_big_recv.at[0], 3),
        ]
        for r in sends:
            r.start()
        big[...] = ctx_ref[...]
        st_big[...] = stats_ref[...]
        for k in range(3):
            rdma(sm32.at[k], sm32.at[k], send_sems.at[15],
                 sm32_recv.at[k], 0).wait_recv()
            rdma(st32.at[k], st32.at[k], send_sems.at[15],
                 st32_recv.at[k], 0).wait_recv()
        rdma(sm128, sm128, send_sems.at[15], sm128_recv.at[0], 0).wait_recv()
        rdma(st128, st128, send_sems.at[15], sm128_recv.at[1], 0).wait_recv()
        for r in sends:
            r.wait_send()

    @pl.when(my_pos == 1)
    def _():
        own32 = ctx_ref.at[pl.ds(0, R0)]
        own128 = ctx_ref.at[pl.ds(R2, SQ - R2)]
        ost32 = stats_ref.at[pl.ds(0, R0)]
        ost128 = stats_ref.at[pl.ds(R2, SQ - R2)]
        sends = []
        for i, dev in enumerate((0, 2, 3)):
            sends += [
                rdma(own32, sm32.at[0], send_sems.at[4 * i + 0],
                     sm32_recv.at[0], dev),
                rdma(own128, sm128, send_sems.at[4 * i + 1],
                     sm128_recv.at[0], dev),
                rdma(ost32, st32.at[0], send_sems.at[4 * i + 2],
                     st32_recv.at[0], dev),
                rdma(ost128, st128, send_sems.at[4 * i + 3],
                     sm128_recv.at[1], dev),
            ]
        for r in sends:
            r.start()
        sm32[0] = ctx_ref[0:R0, :]
        sm128[...] = ctx_ref[R2:SQ, :]
        st32[0] = stats_ref[0:R0, :]
        st128[...] = stats_ref[R2:SQ, :]
        rdma(b_lo, b_lo, send_sems.at[12], big_recv.at[0], 0).wait_recv()
        fwd = rdma(b_lo, b_lo, send_sems.at[12], big_recv.at[0], 2)
        fwd.start()
        rdma(b_hi, b_hi, send_sems.at[15], big_recv.at[1], 0).wait_recv()
        for k in (1, 2):
            rdma(sm32.at[k], sm32.at[k], send_sems.at[15],
                 sm32_recv.at[k], 0).wait_recv()
            rdma(st32.at[k], st32.at[k], send_sems.at[15],
                 st32_recv.at[k], 0).wait_recv()
        rdma(stats_ref, st_big, send_sems.at[15], st_big_recv.at[0], 0).wait_recv()
        fwd.wait_send()
        for r in sends:
            r.wait_send()

    @pl.when(my_pos == 2)
    def _():
        own32 = ctx_ref.at[pl.ds(0, R0)]
        ost32 = stats_ref.at[pl.ds(0, R0)]
        sends = []
        for i, dev in enumerate((0, 1, 3)):
            sends += [
                rdma(own32, sm32.at[1], send_sems.at[2 * i + 0],
                     sm32_recv.at[1], dev),
                rdma(ost32, st32.at[1], send_sems.at[2 * i + 1],
                     st32_recv.at[1], dev),
            ]
        for r in sends:
            r.start()
        sm32[1] = ctx_ref[0:R0, :]
        st32[1] = stats_ref[0:R0, :]
        rdma(b_lo, b_lo, send_sems.at[15], big_recv.at[0], 0).wait_recv()
        rdma(b_hi, b_hi, send_sems.at[15], big_recv.at[1], 0).wait_recv()
        for k in (0, 2):
            rdma(sm32.at[k], sm32.at[k], send_sems.at[15],
                 sm32_recv.at[k], 0).wait_recv()
            rdma(st32.at[k], st32.at[k], send_sems.at[15],
                 st32_recv.at[k], 0).wait_recv()
        rdma(sm128, sm128, send_sems.at[15], sm128_recv.at[0], 0).wait_recv()
        rdma(st128, st128, send_sems.at[15], sm128_recv.at[1], 0).wait_recv()
        rdma(stats_ref, st_big, send_sems.at[15], st_big_recv.at[0], 0).wait_recv()
        for r in sends:
            r.wait_send()

    @pl.when(my_pos == 3)
    def _():
        own32 = ctx_ref.at[pl.ds(0, R0)]
        ost32 = stats_ref.at[pl.ds(0, R0)]
        sends = []
        for i, dev in enumerate((0, 1, 2)):
            sends += [
                rdma(own32, sm32.at[2], send_sems.at[2 * i + 0],
                     sm32_recv.at[2], dev),
                rdma(ost32, st32.at[2], send_sems.at[2 * i + 1],
                     st32_recv.at[2], dev),
            ]
        for r in sends:
            r.start()
        sm32[2] = ctx_ref[0:R0, :]
        st32[2] = stats_ref[0:R0, :]
        rdma(b_hi, b_hi, send_sems.at[12], big_recv.at[1], 0).wait_recv()
        fwd = rdma(b_hi, b_hi, send_sems.at[12], big_recv.at[1], 2)
        fwd.start()
        rdma(b_lo, b_lo, send_sems.at[15], big_recv.at[0], 0).wait_recv()
        for k in (0, 1):
            rdma(sm32.at[k], sm32.at[k], send_sems.at[15],
                 sm32_recv.at[k], 0).wait_recv()
            rdma(st32.at[k], st32.at[k], send_sems.at[15],
                 st32_recv.at[k], 0).wait_recv()
        rdma(sm128, sm128, send_sems.at[15], sm128_recv.at[0], 0).wait_recv()
        rdma(st128, st128, send_sems.at[15], sm128_recv.at[1], 0).wait_recv()
        rdma(stats_ref, st_big, send_sems.at[15], st_big_recv.at[0], 0).wait_recv()
        fwd.wait_send()
        for r in sends:
            r.wait_send()

    for h in range(HQ):
        hs = slice(h * DH, (h + 1) * DH)
        mc = h
        lc = HQ + h

        l0 = st_big[R0:R2, lc:lc + 1]
        out_ref[R0:R2, hs] = big[R0:R2, hs].astype(F32) / l0

        ms = [st_big[0:R0, mc:mc + 1]] + [st32[k, :, mc:mc + 1] for k in range(3)]
        ls = [st_big[0:R0, lc:lc + 1]] + [st32[k, :, lc:lc + 1] for k in range(3)]
        mx = jnp.maximum(jnp.maximum(ms[0], ms[1]), jnp.maximum(ms[2], ms[3]))
        fs = [jnp.exp(m_ - mx) for m_ in ms]
        lsum = sum(l_ * f_ for l_, f_ in zip(ls, fs))
        acc = big[0:R0, hs].astype(F32) * fs[0]
        for k in range(3):
            acc = acc + sm32[k, :, hs].astype(F32) * fs[k + 1]
        out_ref[0:R0, hs] = acc / lsum

        m0 = st_big[R2:SQ, mc:mc + 1]
        m1 = st128[:, mc:mc + 1]
        l0 = st_big[R2:SQ, lc:lc + 1]
        l1 = st128[:, lc:lc + 1]
        mx2 = jnp.maximum(m0, m1)
        f0 = jnp.exp(m0 - mx2)
        f1 = jnp.exp(m1 - mx2)
        acc2 = (big[R2:SQ, hs].astype(F32) * f0
                + sm128[:, hs].astype(F32) * f1)
        out_ref[R2:SQ, hs] = acc2 / (l0 * f0 + l1 * f1)


def kernel(x, Wq, K_ext, V_ext, Wo):
    q2 = jnp.dot(x[0].astype(BF), Wq.astype(BF), preferred_element_type=F32)

    ctx_own, stats_own = pl.pallas_call(
        _attn_body,
        out_shape=[
            jax.ShapeDtypeStruct((SQ, DM), BF),
            jax.ShapeDtypeStruct((SQ, 2 * HQ), F32),
        ],
        in_specs=[pl.BlockSpec(memory_space=pltpu.VMEM)] * 3,
        out_specs=[pl.BlockSpec(memory_space=pltpu.VMEM)] * 2,
        compiler_params=pltpu.CompilerParams(
            vmem_limit_bytes=60 * 1024 * 1024,
        ),
    )(q2, K_ext, V_ext)

    ctxn = pl.pallas_call(
        _merge_body,
        out_shape=jax.ShapeDtypeStruct((SQ, DM), F32),
        in_specs=[pl.BlockSpec(memory_space=pltpu.VMEM)] * 2,
        out_specs=pl.BlockSpec(memory_space=pltpu.VMEM),
        scratch_shapes=[
            pltpu.VMEM((SQ, DM), BF),
            pltpu.VMEM((3, R0, DM), BF),
            pltpu.VMEM((SQ - R2, DM), BF),
            pltpu.VMEM((SQ, 2 * HQ), F32),
            pltpu.VMEM((3, R0, 2 * HQ), F32),
            pltpu.VMEM((SQ - R2, 2 * HQ), F32),
            pltpu.SemaphoreType.DMA((2,)),
            pltpu.SemaphoreType.DMA((3,)),
            pltpu.SemaphoreType.DMA((2,)),
            pltpu.SemaphoreType.DMA((1,)),
            pltpu.SemaphoreType.DMA((3,)),
            pltpu.SemaphoreType.DMA((16,)),
        ],
        compiler_params=pltpu.CompilerParams(
            vmem_limit_bytes=60 * 1024 * 1024,
        ),
    )(ctx_own, stats_own)

    return jnp.dot(ctxn.astype(BF), Wo.astype(BF),
                   preferred_element_type=F32)[None]


# device time: 78914 ns/iter; 1.1685x vs baseline; 1.1685x over previous
import jax
import jax.numpy as jnp
from jax import lax
from jax.experimental import pallas as pl
from jax.experimental.pallas import tpu as pltpu

N_DEV = 4
SQ = 1024
SKV_LOC = 1024
HQ = 8
DH = 128
DM = 1024
SCALE = 0.08838834764831843
NEG = -1e9

HALF = SQ // 2
R0 = 32
R2 = 896
BAND = 128
BF = jnp.bfloat16
F32 = jnp.float32


def _attn_body(q_ref, k_ref, v_ref, ctx_ref, stats_ref):
    my_pos = lax.axis_index("i")

    @pl.when(my_pos == 0)
    def _():
        qi = lax.broadcasted_iota(jnp.int32, (SQ, SKV_LOC), 0)
        kj = lax.broadcasted_iota(jnp.int32, (SQ, SKV_LOC), 1)
        mask = (jnp.abs(qi - kj) <= 128) | (kj < 32) | (qi < 32)
        m_parts = []
        l_parts = []
        for h in range(HQ):
            qh = q_ref[:, h * DH:(h + 1) * DH]
            kh = k_ref[0, :, h, :]
            vh = v_ref[0, :, h, :]
            s = lax.dot_general(qh, kh, (((1,), (1,)), ((), ())),
                                preferred_element_type=F32) * SCALE
            s = jnp.where(mask, s, NEG)
            m = jnp.max(s, axis=1, keepdims=True)
            w = jnp.exp(s - m)
            l = jnp.sum(w, axis=1, keepdims=True)
            ctx_ref[:, h * DH:(h + 1) * DH] = jnp.dot(
                w, vh, preferred_element_type=F32).astype(BF)
            m_parts.append(m)
            l_parts.append(l)
        stats_ref[...] = jnp.concatenate(m_parts + l_parts, axis=1)

    @pl.when(my_pos != 0)
    def _():
        m_parts = []
        l_parts = []
        for h in range(HQ):
            qh = q_ref[0:R0, h * DH:(h + 1) * DH]
            kh = k_ref[0, :, h, :]
            vh = v_ref[0, :, h, :]
            s = lax.dot_general(qh, kh, (((1,), (1,)), ((), ())),
                                preferred_element_type=F32) * SCALE
            m = jnp.max(s, axis=1, keepdims=True)
            w = jnp.exp(s - m)
            l = jnp.sum(w, axis=1, keepdims=True)
            ctx_ref[0:R0, h * DH:(h + 1) * DH] = jnp.dot(
                w, vh, preferred_element_type=F32).astype(BF)
            m_parts.append(m)
            l_parts.append(l)
        stats_ref[0:R0, :] = jnp.concatenate(m_parts + l_parts, axis=1)

    @pl.when(my_pos == 1)
    def _():
        qi = lax.broadcasted_iota(jnp.int32, (SQ - R2, BAND), 0)
        kj = lax.broadcasted_iota(jnp.int32, (SQ - R2, BAND), 1)
        mask = kj <= qi
        m_parts = []
        l_parts = []
        for h in range(HQ):
            qh = q_ref[R2:SQ, h * DH:(h + 1) * DH]
            kh = k_ref[0, 0:BAND, h, :]
            vh = v_ref[0, 0:BAND, h, :]
            s = lax.dot_general(qh, kh, (((1,), (1,)), ((), ())),
                                preferred_element_type=F32) * SCALE
            s = jnp.where(mask, s, NEG)
            m = jnp.max(s, axis=1, keepdims=True)
            w = jnp.exp(s - m)
            l = jnp.sum(w, axis=1, keepdims=True)
            ctx_ref[R2:SQ, h * DH:(h + 1) * DH] = jnp.dot(
                w, vh, preferred_element_type=F32).astype(BF)
            m_parts.append(m)
            l_parts.append(l)
        stats_ref[R2:SQ, :] = jnp.concatenate(m_parts + l_parts, axis=1)


def _merge_body(ctx_ref, stats_ref, out_ref,
                big, sm32, sm128, st_big, st32, st128,
                big_recv, sm32_recv, sm128_recv, st_big_recv, st32_recv,
                send_sems):
    my_pos = lax.axis_index("i")

    def rdma(src, dst, ssem, rsem, dev):
        return pltpu.make_async_remote_copy(
            src_ref=src, dst_ref=dst, send_sem=ssem, recv_sem=rsem,
            device_id=(dev,), device_id_type=pl.DeviceIdType.MESH)

    a_lo = ctx_ref.at[pl.ds(0, HALF)]
    a_hi = ctx_ref.at[pl.ds(HALF, HALF)]
    b_lo = big.at[pl.ds(0, HALF)]
    b_hi = big.at[pl.ds(HALF, HALF)]

    @pl.when(my_pos == 0)
    def _():
        sends = [
            rdma(a_lo, b_lo, send_sems.at[0], big_recv.at[0], 1),
            rdma(a_hi, b_hi, send_sems.at[1], big_recv.at[1], 3),
            rdma(a_hi, b_hi, send_sems.at[2], big_recv.at[1], 1),
            rdma(a_lo, b_lo, send_sems.at[3], big_recv.at[0], 3),
            rdma(stats_ref, st_big, send_sems.at[4], st_big_recv.at[0], 1),
            rdma(stats_ref, st_big, send_sems.at[5], st_big_recv.at[0], 2),
            rdma(stats_ref, st_big, send_sems.at[6], st_big_recv.at[0], 3),
        ]
        for r in sends:
            r.start()
        big[...] = ctx_ref[...]
        st_big[...] = stats_ref[...]
        for k in range(3):
            rdma(sm32.at[k], sm32.at[k], send_sems.at[15],
                 sm32_recv.at[k], 0).wait_recv()
            rdma(st32.at[k], st32.at[k], send_sems.at[15],
                 st32_recv.at[k], 0).wait_recv()
        rdma(sm128, sm128, send_sems.at[15], sm128_recv.at[0], 0).wait_recv()
        rdma(st128, st128, send_sems.at[15], sm128_recv.at[1], 0).wait_recv()
        for r in sends:
            r.wait_send()

    @pl.when(my_pos == 1)
    def _():
        own32 = ctx_ref.at[pl.ds(0, R0)]
        own128 = ctx_ref.at[pl.ds(R2, SQ - R2)]
        ost32 = stats_ref.at[pl.ds(0, R0)]
        ost128 = stats_ref.at[pl.ds(R2, SQ - R2)]
        sends = []
        for i, dev in enumerate((0, 2, 3)):
            sends += [
                rdma(own32, sm32.at[0], send_sems.at[4 * i + 0],
                     sm32_recv.at[0], dev),
                rdma(own128, sm128, send_sems.at[4 * i + 1],
                     sm128_recv.at[0], dev),
                rdma(ost32, st32.at[0], send_sems.at[4 * i + 2],
                     st32_recv.at[0], dev),
                rdma(ost128, st128, send_sems.at[4 * i + 3],
                     sm128_recv.at[1], dev),
            ]
        for r in sends:
            r.start()
        sm32[0] = ctx_ref[0:R0, :]
        sm128[...] = ctx_ref[R2:SQ, :]
        st32[0] = stats_ref[0:R0, :]
        st128[...] = stats_ref[R2:SQ, :]
        rdma(b_lo, b_lo, send_sems.at[12], big_recv.at[0], 0).wait_recv()
        fwd = rdma(b_lo, b_lo, send_sems.at[12], big_recv.at[0], 2)
        fwd.start()
        rdma(b_hi, b_hi, send_sems.at[15], big_recv.at[1], 0).wait_recv()
        for k in (1, 2):
            rdma(sm32.at[k], sm32.at[k], send_sems.at[15],
                 sm32_recv.at[k], 0).wait_recv()
            rdma(st32.at[k], st32.at[k], send_sems.at[15],
                 st32_recv.at[k], 0).wait_recv()
        rdma(stats_ref, st_big, send_sems.at[15], st_big_recv.at[0], 0).wait_recv()
        fwd.wait_send()
        for r in sends:
            r.wait_send()

    @pl.when(my_pos == 2)
    def _():
        own32 = ctx_ref.at[pl.ds(0, R0)]
        ost32 = stats_ref.at[pl.ds(0, R0)]
        sends = []
        for i, dev in enumerate((0, 1, 3)):
            sends += [
                rdma(own32, sm32.at[1], send_sems.at[2 * i + 0],
                     sm32_recv.at[1], dev),
                rdma(ost32, st32.at[1], send_sems.at[2 * i + 1],
                     st32_recv.at[1], dev),
            ]
        for r in sends:
            r.start()
        sm32[1] = ctx_ref[0:R0, :]
        st32[1] = stats_ref[0:R0, :]
        rdma(b_lo, b_lo, send_sems.at[15], big_recv.at[0], 0).wait_recv()
        rdma(b_hi, b_hi, send_sems.at[15], big_recv.at[1], 0).wait_recv()
        for k in (0, 2):
            rdma(sm32.at[k], sm32.at[k], send_sems.at[15],
                 sm32_recv.at[k], 0).wait_recv()
            rdma(st32.at[k], st32.at[k], send_sems.at[15],
                 st32_recv.at[k], 0).wait_recv()
        rdma(sm128, sm128, send_sems.at[15], sm128_recv.at[0], 0).wait_recv()
        rdma(st128, st128, send_sems.at[15], sm128_recv.at[1], 0).wait_recv()
        rdma(stats_ref, st_big, send_sems.at[15], st_big_recv.at[0], 0).wait_recv()
        for r in sends:
            r.wait_send()

    @pl.when(my_pos == 3)
    def _():
        own32 = ctx_ref.at[pl.ds(0, R0)]
        ost32 = stats_ref.at[pl.ds(0, R0)]
        sends = []
        for i, dev in enumerate((0, 1, 2)):
            sends += [
                rdma(own32, sm32.at[2], send_sems.at[2 * i + 0],
                     sm32_recv.at[2], dev),
                rdma(ost32, st32.at[2], send_sems.at[2 * i + 1],
                     st32_recv.at[2], dev),
            ]
        for r in sends:
            r.start()
        sm32[2] = ctx_ref[0:R0, :]
        st32[2] = stats_ref[0:R0, :]
        rdma(b_hi, b_hi, send_sems.at[12], big_recv.at[1], 0).wait_recv()
        fwd = rdma(b_hi, b_hi, send_sems.at[12], big_recv.at[1], 2)
        fwd.start()
        rdma(b_lo, b_lo, send_sems.at[15], big_recv.at[0], 0).wait_recv()
        for k in (0, 1):
            rdma(sm32.at[k], sm32.at[k], send_sems.at[15],
                 sm32_recv.at[k], 0).wait_recv()
            rdma(st32.at[k], st32.at[k], send_sems.at[15],
                 st32_recv.at[k], 0).wait_recv()
        rdma(sm128, sm128, send_sems.at[15], sm128_recv.at[0], 0).wait_recv()
        rdma(st128, st128, send_sems.at[15], sm128_recv.at[1], 0).wait_recv()
        rdma(stats_ref, st_big, send_sems.at[15], st_big_recv.at[0], 0).wait_recv()
        fwd.wait_send()
        for r in sends:
            r.wait_send()

    for h in range(HQ):
        hs = slice(h * DH, (h + 1) * DH)
        mc = h
        lc = HQ + h

        l0 = st_big[R0:R2, lc:lc + 1]
        out_ref[R0:R2, hs] = (big[R0:R2, hs].astype(F32) / l0).astype(BF)

        ms = [st_big[0:R0, mc:mc + 1]] + [st32[k, :, mc:mc + 1] for k in range(3)]
        ls = [st_big[0:R0, lc:lc + 1]] + [st32[k, :, lc:lc + 1] for k in range(3)]
        mx = jnp.maximum(jnp.maximum(ms[0], ms[1]), jnp.maximum(ms[2], ms[3]))
        fs = [jnp.exp(m_ - mx) for m_ in ms]
        lsum = sum(l_ * f_ for l_, f_ in zip(ls, fs))
        acc = big[0:R0, hs].astype(F32) * fs[0]
        for k in range(3):
            acc = acc + sm32[k, :, hs].astype(F32) * fs[k + 1]
        out_ref[0:R0, hs] = (acc / lsum).astype(BF)

        m0 = st_big[R2:SQ, mc:mc + 1]
        m1 = st128[:, mc:mc + 1]
        l0 = st_big[R2:SQ, lc:lc + 1]
        l1 = st128[:, lc:lc + 1]
        mx2 = jnp.maximum(m0, m1)
        f0 = jnp.exp(m0 - mx2)
        f1 = jnp.exp(m1 - mx2)
        acc2 = (big[R2:SQ, hs].astype(F32) * f0
                + sm128[:, hs].astype(F32) * f1)
        out_ref[R2:SQ, hs] = (acc2 / (l0 * f0 + l1 * f1)).astype(BF)


def kernel(x, Wq, K_ext, V_ext, Wo):
    q2 = jnp.dot(x[0], Wq, preferred_element_type=F32)

    ctx_own, stats_own = pl.pallas_call(
        _attn_body,
        out_shape=[
            jax.ShapeDtypeStruct((SQ, DM), BF),
            jax.ShapeDtypeStruct((SQ, 2 * HQ), F32),
        ],
        in_specs=[pl.BlockSpec(memory_space=pltpu.VMEM)] * 3,
        out_specs=[pl.BlockSpec(memory_space=pltpu.VMEM)] * 2,
        compiler_params=pltpu.CompilerParams(
            vmem_limit_bytes=60 * 1024 * 1024,
        ),
    )(q2, K_ext, V_ext)

    ctxn = pl.pallas_call(
        _merge_body,
        out_shape=jax.ShapeDtypeStruct((SQ, DM), BF),
        in_specs=[pl.BlockSpec(memory_space=pltpu.VMEM)] * 2,
        out_specs=pl.BlockSpec(memory_space=pltpu.VMEM),
        scratch_shapes=[
            pltpu.VMEM((SQ, DM), BF),
            pltpu.VMEM((3, R0, DM), BF),
            pltpu.VMEM((SQ - R2, DM), BF),
            pltpu.VMEM((SQ, 2 * HQ), F32),
            pltpu.VMEM((3, R0, 2 * HQ), F32),
            pltpu.VMEM((SQ - R2, 2 * HQ), F32),
            pltpu.SemaphoreType.DMA((2,)),
            pltpu.SemaphoreType.DMA((3,)),
            pltpu.SemaphoreType.DMA((2,)),
            pltpu.SemaphoreType.DMA((1,)),
            pltpu.SemaphoreType.DMA((3,)),
            pltpu.SemaphoreType.DMA((16,)),
        ],
        compiler_params=pltpu.CompilerParams(
            vmem_limit_bytes=60 * 1024 * 1024,
        ),
    )(ctx_own, stats_own)

    return jnp.dot(ctxn, Wo.astype(BF), preferred_element_type=F32)[None]
